# Initial kernel scaffold; baseline (speedup 1.0000x reference)
import functools

import jax
import jax.numpy as jnp
from jax import lax
from jax.experimental import pallas as pl
from jax.experimental.pallas import tpu as pltpu
from jax.experimental.pallas import tpu_sc as plsc

_N = 10000
_E = 320000
_D = 128
_NC = 2
_NS = 16
_NW = _NC * _NS
_EPW = _E // _NW
_K = 80
_NCHUNK = _EPW // _K
_RPT = _N // _NS
_ZROWS = 125


def _agg_body(with_cnt, h_hbm, src_hbm, dst_hbm, *rest):
    if with_cnt:
        (part_hbm, cnt_hbm,
         src_v, dst_v, rows_v, zbuf, ones_v, zcnt, gsem,
         acc_sh, cnt_sh) = rest
    else:
        (part_hbm,
         src_v, dst_v, rows_v, zbuf, gsem, acc_sh) = rest

    c = lax.axis_index("c")
    s = lax.axis_index("s")
    w = s * _NC + c

    def zstore(j, _):
        zbuf[j // 8, pl.ds((j % 8) * 16, 16)] = jnp.zeros((16,), jnp.float32)
        return 0
    lax.fori_loop(0, _ZROWS * _D // 16, zstore, 0)

    row0 = s * _RPT
    for blk in range(_RPT // _ZROWS):
        pltpu.sync_copy(zbuf, acc_sh.at[pl.ds(row0 + blk * _ZROWS, _ZROWS)])
    if with_cnt:
        def ostore(j, _):
            ones_v[j, :] = jnp.ones((16,), jnp.float32)
            zcnt[j, :] = jnp.zeros((16,), jnp.float32)
            return 0
        lax.fori_loop(0, _ZROWS, ostore, 0)
        for blk in range(_RPT // _ZROWS):
            pltpu.sync_copy(zcnt,
                            cnt_sh.at[pl.ds(row0 + blk * _ZROWS, _ZROWS)])
    plsc.subcore_barrier()

    edge0 = w * _EPW

    def chunk(i, _):
        base = pl.multiple_of(edge0 + i * _K, 8)
        pltpu.sync_copy(src_hbm.at[pl.ds(base, _K)], src_v)
        pltpu.sync_copy(dst_hbm.at[pl.ds(base, _K)], dst_v)
        pltpu.async_copy(h_hbm.at[src_v], rows_v, gsem).wait()
        pltpu.sync_copy(rows_v, acc_sh.at[dst_v], add=True)
        if with_cnt:
            pltpu.sync_copy(ones_v.at[pl.ds(0, _K)], cnt_sh.at[dst_v],
                            add=True)
        return 0
    lax.fori_loop(0, _NCHUNK, chunk, 0)

    plsc.subcore_barrier()

    pltpu.sync_copy(acc_sh.at[pl.ds(row0, _RPT)],
                    part_hbm.at[c, pl.ds(row0, _RPT)])
    if with_cnt:
        pltpu.sync_copy(cnt_sh.at[pl.ds(row0, _RPT)],
                        cnt_hbm.at[c, pl.ds(row0, _RPT)])


def _make_agg(with_cnt):
    mesh = plsc.VectorSubcoreMesh(core_axis_name="c", subcore_axis_name="s")
    outs = [jax.ShapeDtypeStruct((_NC, _N, _D), jnp.float32)]
    scratch = [
        pltpu.VMEM((_K,), jnp.int32),
        pltpu.VMEM((_K,), jnp.int32),
        pltpu.VMEM((_K, _D), jnp.float32),
        pltpu.VMEM((_ZROWS, _D), jnp.float32),
    ]
    if with_cnt:
        outs.append(jax.ShapeDtypeStruct((_NC, _N, 16), jnp.float32))
        scratch.append(pltpu.VMEM((_ZROWS, 16), jnp.float32))
        scratch.append(pltpu.VMEM((_ZROWS, 16), jnp.float32))
    scratch.append(pltpu.SemaphoreType.DMA)
    scratch.append(pltpu.VMEM_SHARED((_N, _D), jnp.float32))
    if with_cnt:
        scratch.append(pltpu.VMEM_SHARED((_N, 16), jnp.float32))
    return pl.kernel(
        functools.partial(_agg_body, with_cnt),
        out_type=tuple(outs) if with_cnt else outs[0],
        mesh=mesh,
        scratch_types=scratch,
    )


def _dense_body(part_ref, cnt_ref, h_ref, Wn_ref, Ws_ref, b_ref, out_ref,
                *, relu):
    p = part_ref[0] + part_ref[1]
    cnt = cnt_ref[0, :, 0:1] + cnt_ref[1, :, 0:1]
    mean = p / jnp.maximum(cnt, 1.0)
    acc = jnp.dot(mean, Wn_ref[...], preferred_element_type=jnp.float32)
    acc = acc + b_ref[...]
    acc = acc + jnp.dot(h_ref[...], Ws_ref[...],
                        preferred_element_type=jnp.float32)
    if relu:
        acc = jnp.maximum(acc, 0.0)
    out_ref[...] = acc


def _dense(part, cnt, h, Wn, Ws, b2d, relu):
    R = 1000
    return pl.pallas_call(
        functools.partial(_dense_body, relu=relu),
        grid=(_N // R,),
        in_specs=[
            pl.BlockSpec((_NC, R, _D), lambda i: (0, i, 0)),
            pl.BlockSpec((_NC, R, 16), lambda i: (0, i, 0)),
            pl.BlockSpec((R, _D), lambda i: (i, 0)),
            pl.BlockSpec((_D, _D), lambda i: (0, 0)),
            pl.BlockSpec((_D, _D), lambda i: (0, 0)),
            pl.BlockSpec((1, _D), lambda i: (0, 0)),
        ],
        out_specs=pl.BlockSpec((R, _D), lambda i: (i, 0)),
        out_shape=jax.ShapeDtypeStruct((_N, _D), jnp.float32),
    )(part, cnt, h, Wn, Ws, b2d)


_agg_first = _make_agg(True)
_agg_rest = _make_agg(False)


def kernel(x, edge_index, Wn0, b0, Ws0, Wn1, b1, Ws1, Wn2, b2, Ws2):
    src = edge_index[0]
    dst = edge_index[1]
    part0, cntp = _agg_first(x, src, dst)
    h1 = _dense(part0, cntp, x, Wn0, Ws0, b0.reshape(1, _D), relu=True)
    part1 = _agg_rest(h1, src, dst)
    h2 = _dense(part1, cntp, h1, Wn1, Ws1, b1.reshape(1, _D), relu=True)
    part2 = _agg_rest(h2, src, dst)
    return _dense(part2, cntp, h2, Wn2, Ws2, b2.reshape(1, _D), relu=False)


# speedup vs baseline: 5.0875x; 5.0875x over previous
import functools

import jax
import jax.numpy as jnp
from jax import lax
from jax.experimental import pallas as pl
from jax.experimental.pallas import tpu as pltpu
from jax.experimental.pallas import tpu_sc as plsc

_N = 10000
_E = 320000
_D = 128
_DC = _D + 16
_NC = 2
_NS = 16
_NW = _NC * _NS
_EPW = _E // _NW
_K = 80
_NCHUNK = _EPW // _K
_NP = 10240
_RPT = _NP // _NS
_ZROWS = 64


def _agg_body(dw, h_hbm, src_hbm, dst_hbm, part_hbm,
              src_v, dst_v, rows_v, zbuf, gsem, acc_sh):
    c = lax.axis_index("c")
    s = lax.axis_index("s")
    w = s * _NC + c

    lanes = dw // 16

    def zstore(j, _):
        zbuf[j // lanes, pl.ds((j % lanes) * 16, 16)] = (
            jnp.zeros((16,), jnp.float32))
        return 0
    lax.fori_loop(0, _ZROWS * lanes, zstore, 0)

    row0 = s * _RPT
    for blk in range(_RPT // _ZROWS):
        pltpu.sync_copy(zbuf, acc_sh.at[pl.ds(row0 + blk * _ZROWS, _ZROWS)])
    plsc.subcore_barrier()

    edge0 = w * _EPW

    def chunk(i, _):
        base = pl.multiple_of(edge0 + i * _K, 8)
        pltpu.sync_copy(src_hbm.at[pl.ds(base, _K)], src_v)
        pltpu.sync_copy(dst_hbm.at[pl.ds(base, _K)], dst_v)
        pltpu.async_copy(h_hbm.at[src_v], rows_v, gsem).wait()
        pltpu.sync_copy(rows_v, acc_sh.at[dst_v], add=True)
        return 0
    lax.fori_loop(0, _NCHUNK, chunk, 0)

    plsc.subcore_barrier()

    pltpu.sync_copy(acc_sh.at[pl.ds(row0, _RPT)],
                    part_hbm.at[c, pl.ds(row0, _RPT)])


def _make_agg(dw):
    mesh = plsc.VectorSubcoreMesh(core_axis_name="c", subcore_axis_name="s")
    scratch = [
        pltpu.VMEM((_K,), jnp.int32),
        pltpu.VMEM((_K,), jnp.int32),
        pltpu.VMEM((_K, dw), jnp.float32),
        pltpu.VMEM((_ZROWS, dw), jnp.float32),
        pltpu.SemaphoreType.DMA,
        pltpu.VMEM_SHARED((_NP, dw), jnp.float32),
    ]
    return pl.kernel(
        functools.partial(_agg_body, dw),
        out_type=jax.ShapeDtypeStruct((_NC, _NP, dw), jnp.float32),
        mesh=mesh,
        scratch_types=scratch,
        compiler_params=pltpu.CompilerParams(
            use_tc_tiling_on_sc=(dw % _D == 0)),
    )


def _dense_body(part_ref, cnt_ref, h_ref, Wn_ref, Ws_ref, b_ref, out_ref,
                *, relu):
    p = part_ref[0] + part_ref[1]
    cnt = cnt_ref[0, :, 0:1] + cnt_ref[1, :, 0:1]
    mean = p[:, 0:_D] / jnp.maximum(cnt, 1.0)
    acc = jnp.dot(mean, Wn_ref[...], preferred_element_type=jnp.float32)
    acc = acc + b_ref[...]
    acc = acc + jnp.dot(h_ref[...], Ws_ref[...],
                        preferred_element_type=jnp.float32)
    if relu:
        acc = jnp.maximum(acc, 0.0)
    out_ref[...] = acc


def _dense(part, cnt, h, Wn, Ws, b2d, relu):
    R = 1000
    pw = part.shape[-1]
    return pl.pallas_call(
        functools.partial(_dense_body, relu=relu),
        grid=(_N // R,),
        in_specs=[
            pl.BlockSpec((_NC, R, pw), lambda i: (0, i, 0)),
            pl.BlockSpec((_NC, R, 16), lambda i: (0, i, 0)),
            pl.BlockSpec((R, _D), lambda i: (i, 0)),
            pl.BlockSpec((_D, _D), lambda i: (0, 0)),
            pl.BlockSpec((_D, _D), lambda i: (0, 0)),
            pl.BlockSpec((1, _D), lambda i: (0, 0)),
        ],
        out_specs=pl.BlockSpec((R, _D), lambda i: (i, 0)),
        out_shape=jax.ShapeDtypeStruct((_N, _D), jnp.float32),
    )(part, cnt, h, Wn, Ws, b2d)


_agg_first = _make_agg(_DC)
_agg_rest = _make_agg(_D)


def kernel(x, edge_index, Wn0, b0, Ws0, Wn1, b1, Ws1, Wn2, b2, Ws2):
    src = edge_index[0]
    dst = edge_index[1]
    xc = jnp.concatenate([x, jnp.ones((_N, 16), jnp.float32)], axis=1)
    part0 = _agg_first(xc, src, dst)
    cnt16 = lax.slice(part0, (0, 0, _D), (_NC, _NP, _DC))
    h1 = _dense(part0, cnt16, x, Wn0, Ws0, b0.reshape(1, _D), relu=True)
    part1 = _agg_rest(h1, src, dst)
    h2 = _dense(part1, cnt16, h1, Wn1, Ws1, b1.reshape(1, _D), relu=True)
    part2 = _agg_rest(h2, src, dst)
    return _dense(part2, cnt16, h2, Wn2, Ws2, b2.reshape(1, _D), relu=False)


# speedup vs baseline: 11.3281x; 2.2266x over previous
import functools

import jax
import jax.numpy as jnp
from jax import lax
from jax.experimental import pallas as pl
from jax.experimental.pallas import tpu as pltpu
from jax.experimental.pallas import tpu_sc as plsc

_N = 10000
_E = 320000
_D = 128
_DC = _D + 16
_NC = 2
_NS = 16
_NW = _NC * _NS
_EPW = _E // _NW
_K = 80
_NCHUNK = _EPW // _K
_NP = 10240
_RPT = _NP // _NS
_ZROWS = 64


def _agg_body(dw, zr, h_hbm, src_hbm, dst_hbm, part_hbm,
              src2, dst2, rows2, zbuf, isems, gsems, ssems, acc_sh):
    c = lax.axis_index("c")
    s = lax.axis_index("s")
    w = s * _NC + c

    lanes = dw // 16

    def zstore(j, _):
        zbuf[j // lanes, pl.ds((j % lanes) * 16, 16)] = (
            jnp.zeros((16,), jnp.float32))
        return 0
    lax.fori_loop(0, zr * lanes, zstore, 0)

    row0 = s * _RPT
    for blk in range(_RPT // zr):
        pltpu.sync_copy(zbuf, acc_sh.at[pl.ds(row0 + blk * zr, zr)])
    plsc.subcore_barrier()

    edge0 = w * _EPW

    def ibase(i):
        return pl.multiple_of(edge0 + i * _K, 8)

    def idx_copies(i, q):
        return (pltpu.make_async_copy(src_hbm.at[pl.ds(ibase(i), _K)],
                                      src2.at[q], isems.at[q]),
                pltpu.make_async_copy(dst_hbm.at[pl.ds(ibase(i), _K)],
                                      dst2.at[q], isems.at[q]))

    def gather_copy(q, r):
        return pltpu.make_async_copy(h_hbm.at[src2.at[q]], rows2.at[r],
                                     gsems.at[r])

    def scatter_start(q, r):
        pltpu.async_copy(rows2.at[r], acc_sh.at[dst2.at[q]],
                         ssems.at[r], add=True)

    def scatter_wait(q, r):
        pltpu.make_async_copy(rows2.at[r], acc_sh.at[dst2.at[q]],
                              ssems.at[r]).wait()

    for cp in idx_copies(0, 0):
        cp.start()

    def loop(i, _):
        rb = lax.rem(i, 2)
        nrb = lax.rem(i + 1, 2)
        qb = lax.rem(i, 3)
        qn = lax.rem(i + 1, 3)
        qp = lax.rem(i + 2, 3)

        @pl.when(i >= 2)
        def _():
            scatter_wait(qn, rb)

        @pl.when(i + 1 < _NCHUNK)
        def _():
            for cp in idx_copies(i + 1, qn):
                cp.start()

        for cp in idx_copies(i, qb):
            cp.wait()
        gather_copy(qb, rb).start()

        @pl.when(i >= 1)
        def _():
            gather_copy(qp, nrb).wait()
            scatter_start(qp, nrb)
        return 0
    lax.fori_loop(0, _NCHUNK, loop, 0)

    ql = (_NCHUNK - 1) % 3
    rl = (_NCHUNK - 1) % 2
    gather_copy(ql, rl).wait()
    scatter_start(ql, rl)
    scatter_wait((_NCHUNK - 2) % 3, (_NCHUNK - 2) % 2)
    scatter_wait(ql, rl)

    plsc.subcore_barrier()

    pltpu.sync_copy(acc_sh.at[pl.ds(row0, _RPT)],
                    part_hbm.at[c, pl.ds(row0, _RPT)])


def _make_agg(dw):
    mesh = plsc.VectorSubcoreMesh(core_axis_name="c", subcore_axis_name="s")
    zr = _ZROWS if dw == _D else _ZROWS // 2
    scratch = [
        pltpu.VMEM((3, _K), jnp.int32),
        pltpu.VMEM((3, _K), jnp.int32),
        pltpu.VMEM((2, _K, dw), jnp.float32),
        pltpu.VMEM((zr, dw), jnp.float32),
        pltpu.SemaphoreType.DMA((3,)),
        pltpu.SemaphoreType.DMA((2,)),
        pltpu.SemaphoreType.DMA((2,)),
        pltpu.VMEM_SHARED((_NP, dw), jnp.float32),
    ]
    return pl.kernel(
        functools.partial(_agg_body, dw, zr),
        out_type=jax.ShapeDtypeStruct((_NC, _NP, dw), jnp.float32),
        mesh=mesh,
        scratch_types=scratch,
        compiler_params=pltpu.CompilerParams(
            use_tc_tiling_on_sc=(dw % _D == 0)),
    )


def _dense_body(part_ref, cnt_ref, h_ref, Wn_ref, Ws_ref, b_ref, out_ref,
                *, relu):
    p = part_ref[0] + part_ref[1]
    cnt = cnt_ref[0, :, 0:1] + cnt_ref[1, :, 0:1]
    mean = p[:, 0:_D] / jnp.maximum(cnt, 1.0)
    acc = jnp.dot(mean, Wn_ref[...], preferred_element_type=jnp.float32)
    acc = acc + b_ref[...]
    acc = acc + jnp.dot(h_ref[...], Ws_ref[...],
                        preferred_element_type=jnp.float32)
    if relu:
        acc = jnp.maximum(acc, 0.0)
    out_ref[...] = acc


def _dense(part, cnt, h, Wn, Ws, b2d, relu):
    R = 1000
    pw = part.shape[-1]
    return pl.pallas_call(
        functools.partial(_dense_body, relu=relu),
        grid=(_N // R,),
        in_specs=[
            pl.BlockSpec((_NC, R, pw), lambda i: (0, i, 0)),
            pl.BlockSpec((_NC, R, 16), lambda i: (0, i, 0)),
            pl.BlockSpec((R, _D), lambda i: (i, 0)),
            pl.BlockSpec((_D, _D), lambda i: (0, 0)),
            pl.BlockSpec((_D, _D), lambda i: (0, 0)),
            pl.BlockSpec((1, _D), lambda i: (0, 0)),
        ],
        out_specs=pl.BlockSpec((R, _D), lambda i: (i, 0)),
        out_shape=jax.ShapeDtypeStruct((_N, _D), jnp.float32),
    )(part, cnt, h, Wn, Ws, b2d)


_agg_first = _make_agg(_DC)
_agg_rest = _make_agg(_D)


def kernel(x, edge_index, Wn0, b0, Ws0, Wn1, b1, Ws1, Wn2, b2, Ws2):
    src = edge_index[0]
    dst = edge_index[1]
    xc = jnp.concatenate([x, jnp.ones((_N, 16), jnp.float32)], axis=1)
    part0 = _agg_first(xc, src, dst)
    cnt16 = lax.slice(part0, (0, 0, _D), (_NC, _NP, _DC))
    h1 = _dense(part0, cnt16, x, Wn0, Ws0, b0.reshape(1, _D), relu=True)
    part1 = _agg_rest(h1, src, dst)
    h2 = _dense(part1, cnt16, h1, Wn1, Ws1, b1.reshape(1, _D), relu=True)
    part2 = _agg_rest(h2, src, dst)
    return _dense(part2, cnt16, h2, Wn2, Ws2, b2.reshape(1, _D), relu=False)


# speedup vs baseline: 14.5711x; 1.2863x over previous
import functools

import jax
import jax.numpy as jnp
from jax import lax
from jax.experimental import pallas as pl
from jax.experimental.pallas import tpu as pltpu
from jax.experimental.pallas import tpu_sc as plsc

_N = 10000
_E = 320000
_D = 128
_NC = 2
_NS = 16
_NW = _NC * _NS
_EPW = _E // _NW
_K = 80
_NCHUNK = _EPW // _K
_NP = 10240
_RPT = _NP // _NS
_ZROWS = 64
_R, _Q, _DG, _W = 3, 4, 2, 3


def _agg_body(with_cnt, h_hbm, src_hbm, dst_hbm, *rest):
    if with_cnt:
        (part_hbm, cntf_hbm,
         src2, dst2, rows2, zbuf, ones_k, zflat, isems, gsems, ssems, csems,
         acc_sh, cnt_sh) = rest
    else:
        (part_hbm,
         src2, dst2, rows2, zbuf, isems, gsems, ssems, acc_sh) = rest

    c = lax.axis_index("c")
    s = lax.axis_index("s")
    w = s * _NC + c

    def zstore(j, _):
        zbuf[j // 8, pl.ds((j % 8) * 16, 16)] = jnp.zeros((16,), jnp.float32)
        return 0
    lax.fori_loop(0, _ZROWS * 8, zstore, 0)

    row0 = s * _RPT
    for blk in range(_RPT // _ZROWS):
        pltpu.sync_copy(zbuf, acc_sh.at[pl.ds(row0 + blk * _ZROWS, _ZROWS)])
    if with_cnt:
        def ostore(j, _):
            ones_k[pl.ds(j * 16, 16)] = jnp.ones((16,), jnp.float32)
            return 0
        lax.fori_loop(0, _K // 16, ostore, 0)

        def fstore(j, _):
            zflat[pl.ds(j * 16, 16)] = jnp.zeros((16,), jnp.float32)
            return 0
        lax.fori_loop(0, _RPT // 16, fstore, 0)
        pltpu.sync_copy(zflat, cnt_sh.at[pl.ds(row0, _RPT)])
    plsc.subcore_barrier()

    edge0 = w * _EPW

    def ibase(i):
        return pl.multiple_of(edge0 + i * _K, 8)

    def idx_copies(i, q):
        return (pltpu.make_async_copy(src_hbm.at[pl.ds(ibase(i), _K)],
                                      src2.at[q], isems.at[q]),
                pltpu.make_async_copy(dst_hbm.at[pl.ds(ibase(i), _K)],
                                      dst2.at[q], isems.at[q]))

    def gather_copy(q, r):
        return pltpu.make_async_copy(h_hbm.at[src2.at[q]], rows2.at[r],
                                     gsems.at[r])

    def scatter_start(q, r):
        pltpu.async_copy(rows2.at[r], acc_sh.at[dst2.at[q]],
                         ssems.at[r], add=True)
        if with_cnt:
            pltpu.async_copy(ones_k, cnt_sh.at[dst2.at[q]],
                             csems.at[r], add=True)

    def scatter_wait(q, r):
        pltpu.make_async_copy(rows2.at[r], acc_sh.at[dst2.at[q]],
                              ssems.at[r]).wait()
        if with_cnt:
            pltpu.make_async_copy(ones_k, cnt_sh.at[dst2.at[q]],
                                  csems.at[r]).wait()

    for cp in idx_copies(0, 0):
        cp.start()

    def loop(i, _):
        @pl.when(i >= _W)
        def _():
            scatter_wait(lax.rem(i - _W, _Q), lax.rem(i - _W, _R))

        @pl.when(i + 1 < _NCHUNK)
        def _():
            for cp in idx_copies(i + 1, lax.rem(i + 1, _Q)):
                cp.start()

        for cp in idx_copies(i, lax.rem(i, _Q)):
            cp.wait()
        gather_copy(lax.rem(i, _Q), lax.rem(i, _R)).start()

        @pl.when(i >= _DG)
        def _():
            gather_copy(lax.rem(i - _DG, _Q), lax.rem(i - _DG, _R)).wait()
            scatter_start(lax.rem(i - _DG, _Q), lax.rem(i - _DG, _R))
        return 0
    lax.fori_loop(0, _NCHUNK, loop, 0)

    last = _NCHUNK - 1
    for j in range(last - _DG + 1, last + 1):
        gather_copy(j % _Q, j % _R).wait()
        scatter_start(j % _Q, j % _R)
    for j in range(last - _W + 1, last + 1):
        scatter_wait(j % _Q, j % _R)

    plsc.subcore_barrier()

    pltpu.sync_copy(acc_sh.at[pl.ds(row0, _RPT)],
                    part_hbm.at[c, pl.ds(row0, _RPT)])
    if with_cnt:
        pltpu.sync_copy(cnt_sh.at[pl.ds(row0, _RPT)],
                        cntf_hbm.at[pl.ds(c * _NP + row0, _RPT)])


def _make_agg(with_cnt):
    mesh = plsc.VectorSubcoreMesh(core_axis_name="c", subcore_axis_name="s")
    outs = [jax.ShapeDtypeStruct((_NC, _NP, _D), jnp.float32)]
    scratch = [
        pltpu.VMEM((_Q, _K), jnp.int32),
        pltpu.VMEM((_Q, _K), jnp.int32),
        pltpu.VMEM((_R, _K, _D), jnp.float32),
        pltpu.VMEM((_ZROWS, _D), jnp.float32),
    ]
    if with_cnt:
        outs.append(jax.ShapeDtypeStruct((_NC * _NP,), jnp.float32))
        scratch.append(pltpu.VMEM((_K,), jnp.float32))
        scratch.append(pltpu.VMEM((_RPT,), jnp.float32))
    scratch.append(pltpu.SemaphoreType.DMA((_Q,)))
    scratch.append(pltpu.SemaphoreType.DMA((_R,)))
    scratch.append(pltpu.SemaphoreType.DMA((_R,)))
    if with_cnt:
        scratch.append(pltpu.SemaphoreType.DMA((_R,)))
    scratch.append(pltpu.VMEM_SHARED((_NP, _D), jnp.float32))
    if with_cnt:
        scratch.append(pltpu.VMEM_SHARED((_NP,), jnp.float32))
    return pl.kernel(
        functools.partial(_agg_body, with_cnt),
        out_type=tuple(outs) if with_cnt else outs[0],
        mesh=mesh,
        scratch_types=scratch,
        compiler_params=pltpu.CompilerParams(use_tc_tiling_on_sc=True),
    )


def _dense_body(part_ref, cnt_ref, h_ref, Wn_ref, Ws_ref, b_ref, out_ref,
                *, relu):
    p = part_ref[0] + part_ref[1]
    cnt = cnt_ref[0] + cnt_ref[1]
    mean = p / jnp.maximum(cnt, 1.0)
    acc = jnp.dot(mean, Wn_ref[...], preferred_element_type=jnp.float32)
    acc = acc + b_ref[...]
    acc = acc + jnp.dot(h_ref[...], Ws_ref[...],
                        preferred_element_type=jnp.float32)
    if relu:
        acc = jnp.maximum(acc, 0.0)
    out_ref[...] = acc


def _dense(part, cnt3, h, Wn, Ws, b2d, relu):
    R = 1000
    return pl.pallas_call(
        functools.partial(_dense_body, relu=relu),
        grid=(_N // R,),
        in_specs=[
            pl.BlockSpec((_NC, R, _D), lambda i: (0, i, 0)),
            pl.BlockSpec((_NC, R, 1), lambda i: (0, i, 0)),
            pl.BlockSpec((R, _D), lambda i: (i, 0)),
            pl.BlockSpec((_D, _D), lambda i: (0, 0)),
            pl.BlockSpec((_D, _D), lambda i: (0, 0)),
            pl.BlockSpec((1, _D), lambda i: (0, 0)),
        ],
        out_specs=pl.BlockSpec((R, _D), lambda i: (i, 0)),
        out_shape=jax.ShapeDtypeStruct((_N, _D), jnp.float32),
    )(part, cnt3, h, Wn, Ws, b2d)


_agg_first = _make_agg(True)
_agg_rest = _make_agg(False)


def kernel(x, edge_index, Wn0, b0, Ws0, Wn1, b1, Ws1, Wn2, b2, Ws2):
    src = edge_index[0]
    dst = edge_index[1]
    part0, cntf = _agg_first(x, src, dst)
    cnt3 = cntf.reshape(_NC, _NP, 1)
    h1 = _dense(part0, cnt3, x, Wn0, Ws0, b0.reshape(1, _D), relu=True)
    part1 = _agg_rest(h1, src, dst)
    h2 = _dense(part1, cnt3, h1, Wn1, Ws1, b1.reshape(1, _D), relu=True)
    part2 = _agg_rest(h2, src, dst)
    return _dense(part2, cnt3, h2, Wn2, Ws2, b2.reshape(1, _D), relu=False)


# speedup vs baseline: 14.8903x; 1.0219x over previous
import functools

import jax
import jax.numpy as jnp
from jax import lax
from jax.experimental import pallas as pl
from jax.experimental.pallas import tpu as pltpu
from jax.experimental.pallas import tpu_sc as plsc

_N = 10000
_E = 320000
_D = 128
_NC = 2
_NS = 16
_NW = _NC * _NS
_EPW = _E // _NW
_K = 80
_NCHUNK = _EPW // _K
_NP = 10240
_RPT = _NP // _NS
_ZROWS = 64
_R, _Q, _DG, _W = 3, 4, 2, 3


def _agg_body(with_cnt, h_hbm, src_hbm, dst_hbm, *rest):
    if with_cnt:
        (part_hbm, cntf_hbm,
         src2, dst2, rows2, zbuf, ones_k, zflat, isems, gsems, ssems, csems,
         acc_sh, cnt_sh) = rest
    else:
        (part_hbm,
         src2, dst2, rows2, zbuf, isems, gsems, ssems, acc_sh) = rest

    c = lax.axis_index("c")
    s = lax.axis_index("s")
    w = s * _NC + c

    def zstore(j, _):
        zbuf[j // 8, pl.ds((j % 8) * 16, 16)] = jnp.zeros((16,), jnp.float32)
        return 0
    lax.fori_loop(0, _ZROWS * 8, zstore, 0)

    row0 = s * _RPT
    for blk in range(_RPT // _ZROWS):
        pltpu.sync_copy(zbuf, acc_sh.at[pl.ds(row0 + blk * _ZROWS, _ZROWS)])
    if with_cnt:
        def ostore(j, _):
            ones_k[pl.ds(j * 16, 16)] = jnp.ones((16,), jnp.float32)
            return 0
        lax.fori_loop(0, _K // 16, ostore, 0)

        def fstore(j, _):
            zflat[pl.ds(j * 16, 16)] = jnp.zeros((16,), jnp.float32)
            return 0
        lax.fori_loop(0, _RPT // 16, fstore, 0)
        pltpu.sync_copy(zflat, cnt_sh.at[pl.ds(row0, _RPT)])
    plsc.subcore_barrier()

    edge0 = w * _EPW

    def ibase(i):
        return pl.multiple_of(edge0 + i * _K, 8)

    def idx_copies(i, q):
        return (pltpu.make_async_copy(src_hbm.at[pl.ds(ibase(i), _K)],
                                      src2.at[q], isems.at[q]),
                pltpu.make_async_copy(dst_hbm.at[pl.ds(ibase(i), _K)],
                                      dst2.at[q], isems.at[q]))

    def gather_copy(q, r):
        return pltpu.make_async_copy(h_hbm.at[src2.at[q]], rows2.at[r],
                                     gsems.at[r])

    def scatter_start(q, r):
        pltpu.async_copy(rows2.at[r], acc_sh.at[dst2.at[q]],
                         ssems.at[r], add=True)
        if with_cnt:
            pltpu.async_copy(ones_k, cnt_sh.at[dst2.at[q]],
                             csems.at[r], add=True)

    def scatter_wait(q, r):
        pltpu.make_async_copy(rows2.at[r], acc_sh.at[dst2.at[q]],
                              ssems.at[r]).wait()
        if with_cnt:
            pltpu.make_async_copy(ones_k, cnt_sh.at[dst2.at[q]],
                                  csems.at[r]).wait()

    for cp in idx_copies(0, 0):
        cp.start()

    def loop(i, _):
        @pl.when(i >= _W)
        def _():
            scatter_wait(lax.rem(i - _W, _Q), lax.rem(i - _W, _R))

        @pl.when(i + 1 < _NCHUNK)
        def _():
            for cp in idx_copies(i + 1, lax.rem(i + 1, _Q)):
                cp.start()

        for cp in idx_copies(i, lax.rem(i, _Q)):
            cp.wait()
        gather_copy(lax.rem(i, _Q), lax.rem(i, _R)).start()

        @pl.when(i >= _DG)
        def _():
            gather_copy(lax.rem(i - _DG, _Q), lax.rem(i - _DG, _R)).wait()
            scatter_start(lax.rem(i - _DG, _Q), lax.rem(i - _DG, _R))
        return 0
    lax.fori_loop(0, _NCHUNK, loop, 0)

    last = _NCHUNK - 1
    for j in range(last - _DG + 1, last + 1):
        gather_copy(j % _Q, j % _R).wait()
        scatter_start(j % _Q, j % _R)
    for j in range(last - _W + 1, last + 1):
        scatter_wait(j % _Q, j % _R)

    plsc.subcore_barrier()

    pltpu.sync_copy(acc_sh.at[pl.ds(row0, _RPT)],
                    part_hbm.at[c, pl.ds(row0, _RPT)])
    if with_cnt:
        pltpu.sync_copy(cnt_sh.at[pl.ds(row0, _RPT)],
                        cntf_hbm.at[pl.ds(c * _NP + row0, _RPT)])


def _make_agg(with_cnt):
    mesh = plsc.VectorSubcoreMesh(core_axis_name="c", subcore_axis_name="s")
    outs = [jax.ShapeDtypeStruct((_NC, _NP, _D), jnp.float32)]
    scratch = [
        pltpu.VMEM((_Q, _K), jnp.int32),
        pltpu.VMEM((_Q, _K), jnp.int32),
        pltpu.VMEM((_R, _K, _D), jnp.float32),
        pltpu.VMEM((_ZROWS, _D), jnp.float32),
    ]
    if with_cnt:
        outs.append(jax.ShapeDtypeStruct((_NC * _NP,), jnp.float32))
        scratch.append(pltpu.VMEM((_K,), jnp.float32))
        scratch.append(pltpu.VMEM((_RPT,), jnp.float32))
    scratch.append(pltpu.SemaphoreType.DMA((_Q,)))
    scratch.append(pltpu.SemaphoreType.DMA((_R,)))
    scratch.append(pltpu.SemaphoreType.DMA((_R,)))
    if with_cnt:
        scratch.append(pltpu.SemaphoreType.DMA((_R,)))
    scratch.append(pltpu.VMEM_SHARED((_NP, _D), jnp.float32))
    if with_cnt:
        scratch.append(pltpu.VMEM_SHARED((_NP,), jnp.float32))
    return pl.kernel(
        functools.partial(_agg_body, with_cnt),
        out_type=tuple(outs) if with_cnt else outs[0],
        mesh=mesh,
        scratch_types=scratch,
        compiler_params=pltpu.CompilerParams(use_tc_tiling_on_sc=True),
    )


def _dense_body(part_ref, cnt_ref, h_ref, Wn_ref, Ws_ref, b_ref, out_ref,
                *, relu):
    p = part_ref[0] + part_ref[1]
    cnt = cnt_ref[0] + cnt_ref[1]
    mean = p / jnp.maximum(cnt, 1.0)
    acc = jnp.dot(mean, Wn_ref[...], preferred_element_type=jnp.float32)
    acc = acc + b_ref[...]
    acc = acc + jnp.dot(h_ref[...], Ws_ref[...],
                        preferred_element_type=jnp.float32)
    if relu:
        acc = jnp.maximum(acc, 0.0)
    out_ref[...] = acc


def _dense(part, cnt3, h, Wn, Ws, b2d, relu):
    R = 2000
    return pl.pallas_call(
        functools.partial(_dense_body, relu=relu),
        grid=(_N // R,),
        in_specs=[
            pl.BlockSpec((_NC, R, _D), lambda i: (0, i, 0)),
            pl.BlockSpec((_NC, R, 1), lambda i: (0, i, 0)),
            pl.BlockSpec((R, _D), lambda i: (i, 0)),
            pl.BlockSpec((_D, _D), lambda i: (0, 0)),
            pl.BlockSpec((_D, _D), lambda i: (0, 0)),
            pl.BlockSpec((1, _D), lambda i: (0, 0)),
        ],
        out_specs=pl.BlockSpec((R, _D), lambda i: (i, 0)),
        out_shape=jax.ShapeDtypeStruct((_N, _D), jnp.float32),
    )(part, cnt3, h, Wn, Ws, b2d)


_agg_first = _make_agg(True)
_agg_rest = _make_agg(False)


def kernel(x, edge_index, Wn0, b0, Ws0, Wn1, b1, Ws1, Wn2, b2, Ws2):
    src = edge_index[0]
    dst = edge_index[1]
    part0, cntf = _agg_first(x, src, dst)
    cnt3 = cntf.reshape(_NC, _NP, 1)
    h1 = _dense(part0, cnt3, x, Wn0, Ws0, b0.reshape(1, _D), relu=True)
    part1 = _agg_rest(h1, src, dst)
    h2 = _dense(part1, cnt3, h1, Wn1, Ws1, b1.reshape(1, _D), relu=True)
    part2 = _agg_rest(h2, src, dst)
    return _dense(part2, cnt3, h2, Wn2, Ws2, b2.reshape(1, _D), relu=False)


# speedup vs baseline: 15.6521x; 1.0512x over previous
import functools

import jax
import jax.numpy as jnp
from jax import lax
from jax.experimental import pallas as pl
from jax.experimental.pallas import tpu as pltpu
from jax.experimental.pallas import tpu_sc as plsc

_N = 10000
_E = 320000
_D = 128
_NC = 2
_NS = 16
_NW = _NC * _NS
_EPW = _E // _NW
_K = 80
_NCHUNK = _EPW // _K
_NP = 10240
_RPT = _NP // _NS
_ZROWS = 32
_R, _Q, _DG, _W = 4, 5, 3, 4


def _agg_body(with_cnt, h_hbm, src_hbm, dst_hbm, *rest):
    if with_cnt:
        (part_hbm, cntf_hbm,
         src2, dst2, rows2, zbuf, ones_k, zflat, isems, gsems, ssems, csems,
         acc_sh, cnt_sh) = rest
    else:
        (part_hbm,
         src2, dst2, rows2, zbuf, isems, gsems, ssems, acc_sh) = rest

    c = lax.axis_index("c")
    s = lax.axis_index("s")
    w = s * _NC + c

    def zstore(j, _):
        zbuf[j // 8, pl.ds((j % 8) * 16, 16)] = jnp.zeros((16,), jnp.float32)
        return 0
    lax.fori_loop(0, _ZROWS * 8, zstore, 0)

    row0 = s * _RPT
    for blk in range(_RPT // _ZROWS):
        pltpu.sync_copy(zbuf, acc_sh.at[pl.ds(row0 + blk * _ZROWS, _ZROWS)])
    if with_cnt:
        def ostore(j, _):
            ones_k[pl.ds(j * 16, 16)] = jnp.ones((16,), jnp.float32)
            return 0
        lax.fori_loop(0, _K // 16, ostore, 0)

        def fstore(j, _):
            zflat[pl.ds(j * 16, 16)] = jnp.zeros((16,), jnp.float32)
            return 0
        lax.fori_loop(0, _RPT // 16, fstore, 0)
        pltpu.sync_copy(zflat, cnt_sh.at[pl.ds(row0, _RPT)])
    plsc.subcore_barrier()

    edge0 = w * _EPW

    def ibase(i):
        return pl.multiple_of(edge0 + i * _K, 8)

    def idx_copies(i, q):
        return (pltpu.make_async_copy(src_hbm.at[pl.ds(ibase(i), _K)],
                                      src2.at[q], isems.at[q]),
                pltpu.make_async_copy(dst_hbm.at[pl.ds(ibase(i), _K)],
                                      dst2.at[q], isems.at[q]))

    def gather_copy(q, r):
        return pltpu.make_async_copy(h_hbm.at[src2.at[q]], rows2.at[r],
                                     gsems.at[r])

    def scatter_start(q, r):
        pltpu.async_copy(rows2.at[r], acc_sh.at[dst2.at[q]],
                         ssems.at[r], add=True)
        if with_cnt:
            pltpu.async_copy(ones_k, cnt_sh.at[dst2.at[q]],
                             csems.at[r], add=True)

    def scatter_wait(q, r):
        pltpu.make_async_copy(rows2.at[r], acc_sh.at[dst2.at[q]],
                              ssems.at[r]).wait()
        if with_cnt:
            pltpu.make_async_copy(ones_k, cnt_sh.at[dst2.at[q]],
                                  csems.at[r]).wait()

    for cp in idx_copies(0, 0):
        cp.start()

    def loop(i, _):
        @pl.when(i >= _W)
        def _():
            scatter_wait(lax.rem(i - _W, _Q), lax.rem(i - _W, _R))

        @pl.when(i + 1 < _NCHUNK)
        def _():
            for cp in idx_copies(i + 1, lax.rem(i + 1, _Q)):
                cp.start()

        for cp in idx_copies(i, lax.rem(i, _Q)):
            cp.wait()
        gather_copy(lax.rem(i, _Q), lax.rem(i, _R)).start()

        @pl.when(i >= _DG)
        def _():
            gather_copy(lax.rem(i - _DG, _Q), lax.rem(i - _DG, _R)).wait()
            scatter_start(lax.rem(i - _DG, _Q), lax.rem(i - _DG, _R))
        return 0
    lax.fori_loop(0, _NCHUNK, loop, 0)

    last = _NCHUNK - 1
    for j in range(last - _DG + 1, last + 1):
        gather_copy(j % _Q, j % _R).wait()
        scatter_start(j % _Q, j % _R)
    for j in range(last - _W + 1, last + 1):
        scatter_wait(j % _Q, j % _R)

    plsc.subcore_barrier()

    pltpu.sync_copy(acc_sh.at[pl.ds(row0, _RPT)],
                    part_hbm.at[c, pl.ds(row0, _RPT)])
    if with_cnt:
        pltpu.sync_copy(cnt_sh.at[pl.ds(row0, _RPT)],
                        cntf_hbm.at[pl.ds(c * _NP + row0, _RPT)])


def _make_agg(with_cnt):
    mesh = plsc.VectorSubcoreMesh(core_axis_name="c", subcore_axis_name="s")
    outs = [jax.ShapeDtypeStruct((_NC, _NP, _D), jnp.float32)]
    scratch = [
        pltpu.VMEM((_Q, _K), jnp.int32),
        pltpu.VMEM((_Q, _K), jnp.int32),
        pltpu.VMEM((_R, _K, _D), jnp.float32),
        pltpu.VMEM((_ZROWS, _D), jnp.float32),
    ]
    if with_cnt:
        outs.append(jax.ShapeDtypeStruct((_NC * _NP,), jnp.float32))
        scratch.append(pltpu.VMEM((_K,), jnp.float32))
        scratch.append(pltpu.VMEM((_RPT,), jnp.float32))
    scratch.append(pltpu.SemaphoreType.DMA((_Q,)))
    scratch.append(pltpu.SemaphoreType.DMA((_R,)))
    scratch.append(pltpu.SemaphoreType.DMA((_R,)))
    if with_cnt:
        scratch.append(pltpu.SemaphoreType.DMA((_R,)))
    scratch.append(pltpu.VMEM_SHARED((_NP, _D), jnp.float32))
    if with_cnt:
        scratch.append(pltpu.VMEM_SHARED((_NP,), jnp.float32))
    return pl.kernel(
        functools.partial(_agg_body, with_cnt),
        out_type=tuple(outs) if with_cnt else outs[0],
        mesh=mesh,
        scratch_types=scratch,
        compiler_params=pltpu.CompilerParams(use_tc_tiling_on_sc=True),
    )


def _dense_body(part_ref, cnt_ref, h_ref, Wn_ref, Ws_ref, b_ref, out_ref,
                *, relu):
    p = part_ref[0] + part_ref[1]
    cnt = cnt_ref[0] + cnt_ref[1]
    mean = p / jnp.maximum(cnt, 1.0)
    acc = jnp.dot(mean, Wn_ref[...], preferred_element_type=jnp.float32)
    acc = acc + b_ref[...]
    acc = acc + jnp.dot(h_ref[...], Ws_ref[...],
                        preferred_element_type=jnp.float32)
    if relu:
        acc = jnp.maximum(acc, 0.0)
    out_ref[...] = acc


def _dense(part, cnt3, h, Wn, Ws, b2d, relu):
    R = 2000
    return pl.pallas_call(
        functools.partial(_dense_body, relu=relu),
        grid=(_N // R,),
        in_specs=[
            pl.BlockSpec((_NC, R, _D), lambda i: (0, i, 0)),
            pl.BlockSpec((_NC, R, 1), lambda i: (0, i, 0)),
            pl.BlockSpec((R, _D), lambda i: (i, 0)),
            pl.BlockSpec((_D, _D), lambda i: (0, 0)),
            pl.BlockSpec((_D, _D), lambda i: (0, 0)),
            pl.BlockSpec((1, _D), lambda i: (0, 0)),
        ],
        out_specs=pl.BlockSpec((R, _D), lambda i: (i, 0)),
        out_shape=jax.ShapeDtypeStruct((_N, _D), jnp.float32),
    )(part, cnt3, h, Wn, Ws, b2d)


_agg_first = _make_agg(True)
_agg_rest = _make_agg(False)


def kernel(x, edge_index, Wn0, b0, Ws0, Wn1, b1, Ws1, Wn2, b2, Ws2):
    src = edge_index[0]
    dst = edge_index[1]
    part0, cntf = _agg_first(x, src, dst)
    cnt3 = cntf.reshape(_NC, _NP, 1)
    h1 = _dense(part0, cnt3, x, Wn0, Ws0, b0.reshape(1, _D), relu=True)
    part1 = _agg_rest(h1, src, dst)
    h2 = _dense(part1, cnt3, h1, Wn1, Ws1, b1.reshape(1, _D), relu=True)
    part2 = _agg_rest(h2, src, dst)
    return _dense(part2, cnt3, h2, Wn2, Ws2, b2.reshape(1, _D), relu=False)
